# baseline (speedup 1.0000x reference)
import functools

import jax
import jax.numpy as jnp
from jax import lax
from jax.experimental import pallas as pl
from jax.experimental.pallas import tpu as pltpu
from jax.experimental.pallas import tpu_sc as plsc

D_MODEL = 128
N_ROWS = 4096 * 200
NC, NS = 2, 16
NW = NC * NS
ROWS_PER_W = N_ROWS // NW
CHUNK = 128
NCHUNK = ROWS_PER_W // CHUNK
NPAIR = NCHUNK // 2
MAX_ROWS = 398
HALF = CHUNK // 2
VROW_UNROLL = 4


@functools.partial(
    pl.kernel,
    out_type=jax.ShapeDtypeStruct((N_ROWS, D_MODEL), jnp.float32),
    mesh=plsc.VectorSubcoreMesh(core_axis_name="c", subcore_axis_name="s"),
    scratch_types=[
        pltpu.VMEM((NCHUNK, CHUNK), jnp.int32),
        pltpu.VMEM((MAX_ROWS, D_MODEL), jnp.float32),
        pltpu.VMEM_SHARED((MAX_ROWS, D_MODEL), jnp.float32),
        [pltpu.VMEM((CHUNK, D_MODEL), jnp.float32) for _ in range(2)],
        [pltpu.VMEM((HALF, D_MODEL), jnp.float32) for _ in range(2)],
        [pltpu.SemaphoreType.DMA for _ in range(2)],
        [pltpu.SemaphoreType.DMA for _ in range(2)],
        [pltpu.SemaphoreType.DMA for _ in range(2)],
    ],
)
def _gather_rows(idx_hbm, pe_hbm, out_hbm, idx_v, table_v, table_sh,
                 cb_rows, vx_rows, gsems, cb_ssems, vx_ssems):
    wid = lax.axis_index("s") * NC + lax.axis_index("c")
    base = wid * ROWS_PER_W

    @pl.when(lax.axis_index("s") == 0)
    def _():
        pltpu.sync_copy(pe_hbm, table_sh)

    pltpu.sync_copy(pe_hbm, table_v)
    pltpu.sync_copy(idx_hbm.at[wid], idx_v)
    plsc.subcore_barrier()

    def out_slice(j, off=0, rows=CHUNK):
        return out_hbm.at[pl.ds(base + j * CHUNK + off, rows)]

    def cb_gather(t, a):
        pltpu.async_copy(table_sh.at[idx_v.at[2 * t]], cb_rows[a], gsems[a])

    def cb_gather_wait(a):
        pltpu.make_async_copy(table_sh.at[idx_v.at[0]], cb_rows[a], gsems[a]).wait()

    def cb_scatter(t, a):
        pltpu.async_copy(cb_rows[a], out_slice(2 * t), cb_ssems[a])

    def cb_scatter_wait(a):
        pltpu.make_async_copy(cb_rows[a], out_slice(0), cb_ssems[a]).wait()

    def vx_fill(j, h, buf):
        def row_group(it, carry):
            i0 = it * 16
            idxv = idx_v[j, pl.ds(h * HALF + i0, 16)]
            for rr in range(16):
                i = i0 + rr
                r = idxv[rr]
                for k in range(D_MODEL // 16):
                    buf[i, pl.ds(16 * k, 16)] = table_v[r, pl.ds(16 * k, 16)]
            return carry

        lax.fori_loop(0, HALF // 16, row_group, 0)

    def vx_chunk(j, wait_pred):
        for h in range(2):
            def _wait(h=h):
                pltpu.make_async_copy(
                    vx_rows[h], out_slice(0, rows=HALF), vx_ssems[h]).wait()

            if wait_pred is True:
                _wait()
            else:
                pl.when(wait_pred)(_wait)

            vx_fill(j, h, vx_rows[h])
            pltpu.async_copy(
                vx_rows[h], out_slice(j, off=h * HALF, rows=HALF), vx_ssems[h])

    cb_gather(0, 0)
    cb_gather(1, 1)

    def step(q, carry):
        for p in range(2):
            t = 2 * q + p
            cb_gather_wait(p)
            cb_scatter(t, p)
            wait_pred = (q > 0) if p == 0 else True
            vx_chunk(2 * t + 1, wait_pred)

            @pl.when(t + 2 < NPAIR)
            def _():
                cb_scatter_wait(p)
                cb_gather(t + 2, p)

        return carry

    lax.fori_loop(0, NPAIR // 2, step, 0)

    for a in range(2):
        cb_scatter_wait(a)
    for h in range(2):
        pltpu.make_async_copy(vx_rows[h], out_slice(0, rows=HALF), vx_ssems[h]).wait()


def kernel(days, pe):
    idx = days.reshape(NW, NCHUNK, CHUNK)
    out = _gather_rows(idx, pe)
    return out.reshape(days.shape[0], days.shape[1], D_MODEL)


# speedup vs baseline: 1.5843x; 1.5843x over previous
import functools

import jax
import jax.numpy as jnp
from jax import lax
from jax.experimental import pallas as pl
from jax.experimental.pallas import tpu as pltpu
from jax.experimental.pallas import tpu_sc as plsc

D_MODEL = 128
N_ROWS = 4096 * 200
NC, NS = 2, 16
NW = NC * NS
ROWS_PER_W = N_ROWS // NW
CHUNK = 128
NCHUNK = ROWS_PER_W // CHUNK
NPAIR = NCHUNK // 2
MAX_ROWS = 398
HALF = CHUNK // 2
VROW_UNROLL = 4


@functools.partial(
    pl.kernel,
    out_type=jax.ShapeDtypeStruct((N_ROWS, D_MODEL), jnp.float32),
    mesh=plsc.VectorSubcoreMesh(core_axis_name="c", subcore_axis_name="s"),
    scratch_types=[
        pltpu.VMEM((NCHUNK, CHUNK), jnp.int32),
        pltpu.VMEM((MAX_ROWS, D_MODEL), jnp.float32),
        pltpu.VMEM_SHARED((MAX_ROWS, D_MODEL), jnp.float32),
        [pltpu.VMEM((CHUNK, D_MODEL), jnp.float32) for _ in range(2)],
        [pltpu.VMEM((HALF, D_MODEL), jnp.float32) for _ in range(2)],
        [pltpu.SemaphoreType.DMA for _ in range(2)],
        [pltpu.SemaphoreType.DMA for _ in range(2)],
        [pltpu.SemaphoreType.DMA for _ in range(2)],
    ],
)
def _gather_rows(idx_hbm, pe_hbm, out_hbm, idx_v, table_v, table_sh,
                 cb_rows, vx_rows, gsems, cb_ssems, vx_ssems):
    wid = lax.axis_index("s") * NC + lax.axis_index("c")
    base = wid * ROWS_PER_W

    @pl.when(lax.axis_index("s") == 0)
    def _():
        pltpu.sync_copy(pe_hbm, table_sh)

    pltpu.sync_copy(pe_hbm, table_v)
    pltpu.sync_copy(idx_hbm.at[wid], idx_v)
    plsc.subcore_barrier()

    def out_slice(j, off=0, rows=CHUNK):
        return out_hbm.at[pl.ds(base + j * CHUNK + off, rows)]

    def cb_gather(t, a):
        pltpu.async_copy(table_sh.at[idx_v.at[2 * t]], cb_rows[a], gsems[a])

    def cb_gather_wait(a):
        pltpu.make_async_copy(table_sh.at[idx_v.at[0]], cb_rows[a], gsems[a]).wait()

    def cb_scatter(t, a):
        pltpu.async_copy(cb_rows[a], out_slice(2 * t), cb_ssems[a])

    def cb_scatter_wait(a):
        pltpu.make_async_copy(cb_rows[a], out_slice(0), cb_ssems[a]).wait()

    def vx_fill(j, h, buf):
        @plsc.parallel_loop(0, HALF // 16, step=1, carry=jnp.int32(0))
        def row_group(it, carry):
            i0 = it * 16
            idxv = idx_v[j, pl.ds(h * HALF + i0, 16)]
            for rr in range(16):
                r = idxv[rr]
                for k in range(D_MODEL // 16):
                    buf[i0 + rr, pl.ds(16 * k, 16)] = table_v[r, pl.ds(16 * k, 16)]
            return carry

    def vx_chunk(j, wait_pred):
        for h in range(2):
            def _wait(h=h):
                pltpu.make_async_copy(
                    vx_rows[h], out_slice(0, rows=HALF), vx_ssems[h]).wait()

            if wait_pred is True:
                _wait()
            else:
                pl.when(wait_pred)(_wait)

            vx_fill(j, h, vx_rows[h])
            pltpu.async_copy(
                vx_rows[h], out_slice(j, off=h * HALF, rows=HALF), vx_ssems[h])

    cb_gather(0, 0)
    cb_gather(1, 1)

    def step(q, carry):
        for p in range(2):
            t = 2 * q + p
            cb_gather_wait(p)
            cb_scatter(t, p)
            wait_pred = (q > 0) if p == 0 else True
            vx_chunk(2 * t + 1, wait_pred)

            @pl.when(t + 2 < NPAIR)
            def _():
                cb_scatter_wait(p)
                cb_gather(t + 2, p)

        return carry

    lax.fori_loop(0, NPAIR // 2, step, 0)

    for a in range(2):
        cb_scatter_wait(a)
    for h in range(2):
        pltpu.make_async_copy(vx_rows[h], out_slice(0, rows=HALF), vx_ssems[h]).wait()


def kernel(days, pe):
    idx = days.reshape(NW, NCHUNK, CHUNK)
    out = _gather_rows(idx, pe)
    return out.reshape(days.shape[0], days.shape[1], D_MODEL)


# speedup vs baseline: 2.4048x; 1.5180x over previous
import functools

import jax
import jax.numpy as jnp
from jax import lax
from jax.experimental import pallas as pl
from jax.experimental.pallas import tpu as pltpu
from jax.experimental.pallas import tpu_sc as plsc

D_MODEL = 128
N_ROWS = 4096 * 200
NC, NS = 2, 16
NW = NC * NS
ROWS_PER_W = N_ROWS // NW
CHUNK = 128
NCHUNK = ROWS_PER_W // CHUNK
NBUF = 5
MAX_ROWS = 398


@functools.partial(
    pl.kernel,
    out_type=jax.ShapeDtypeStruct((N_ROWS, D_MODEL), jnp.float32),
    mesh=plsc.VectorSubcoreMesh(core_axis_name="c", subcore_axis_name="s"),
    scratch_types=[
        pltpu.VMEM((NCHUNK, CHUNK), jnp.int32),
        pltpu.VMEM_SHARED((MAX_ROWS, D_MODEL), jnp.float32),
        [pltpu.VMEM((CHUNK, D_MODEL), jnp.float32) for _ in range(NBUF)],
        [pltpu.SemaphoreType.DMA for _ in range(NBUF)],
        [pltpu.SemaphoreType.DMA for _ in range(NBUF)],
    ],
)
def _gather_rows(idx_hbm, pe_hbm, out_hbm, idx_v, table_sh, rows, gsems, ssems):
    wid = lax.axis_index("s") * NC + lax.axis_index("c")
    base = wid * ROWS_PER_W

    @pl.when(lax.axis_index("s") == 0)
    def _():
        pltpu.sync_copy(pe_hbm, table_sh)

    pltpu.sync_copy(idx_hbm.at[wid], idx_v)
    plsc.subcore_barrier()

    def table_src(b):
        return table_sh

    for b in range(NBUF - 1):
        pltpu.async_copy(table_src(b).at[idx_v.at[b]], rows[b], gsems[b])

    def out_slice(j):
        return out_hbm.at[pl.ds(base + j * CHUNK, CHUNK)]

    def step(j0, carry):
        for b in range(NBUF):
            j = j0 * NBUF + b
            pltpu.make_async_copy(table_src(b).at[idx_v.at[j]], rows[b], gsems[b]).wait()
            pltpu.async_copy(rows[b], out_slice(j), ssems[b])
            g = j + NBUF - 1
            bg = (b - 1) % NBUF

            @pl.when(jnp.logical_and(g < NCHUNK, j > 0))
            def _():
                pltpu.make_async_copy(rows[bg], out_slice(g - NBUF), ssems[bg]).wait()
                pltpu.async_copy(table_src(bg).at[idx_v.at[g]], rows[bg], gsems[bg])

            @pl.when(jnp.logical_and(g < NCHUNK, j == 0))
            def _():
                pltpu.async_copy(table_src(bg).at[idx_v.at[g]], rows[bg], gsems[bg])

        return carry

    lax.fori_loop(0, NCHUNK // NBUF, step, 0)

    for b in range(NBUF):
        j = NCHUNK - NBUF + b
        pltpu.make_async_copy(rows[b], out_slice(j), ssems[b]).wait()


def kernel(days, pe):
    idx = days.reshape(NW, NCHUNK, CHUNK)
    out = _gather_rows(idx, pe)
    return out.reshape(days.shape[0], days.shape[1], D_MODEL)
